# baseline (speedup 1.0000x reference)
import functools

import jax
import jax.numpy as jnp
from jax.experimental import pallas as pl
from jax.experimental.pallas import tpu as pltpu

EPS = 1e-5


def _qemb_ln_kernel(x_ref, tag_ref, table_ref, gamma_ref, beta_ref, out_ref):
    x = x_ref[...]
    t = tag_ref[...]
    row0 = table_ref[0, :][None, :]
    row1 = table_ref[1, :][None, :]
    emb = jnp.where(t > 0.5, row1, row0)
    y = x + emb
    d = y.shape[-1]
    mean = jnp.sum(y, axis=-1, keepdims=True) * (1.0 / d)
    yc = y - mean
    var = jnp.sum(yc * yc, axis=-1, keepdims=True) * (1.0 / d)
    normed = yc * jax.lax.rsqrt(var + EPS)
    out_ref[...] = normed * gamma_ref[...] + beta_ref[...]


@functools.partial(jax.jit, static_argnames=())
def kernel(x, tags, table, gamma, beta):
    B, S, D = x.shape
    N = B * S
    RB = 512
    x2 = x.reshape(N, D)
    tagf = tags.reshape(N, 1).astype(jnp.float32)
    gamma2 = gamma.reshape(1, D)
    beta2 = beta.reshape(1, D)

    out = pl.pallas_call(
        _qemb_ln_kernel,
        grid=(N // RB,),
        in_specs=[
            pl.BlockSpec((RB, D), lambda i: (i, 0)),
            pl.BlockSpec((RB, 1), lambda i: (i, 0)),
            pl.BlockSpec((2, D), lambda i: (0, 0)),
            pl.BlockSpec((1, D), lambda i: (0, 0)),
            pl.BlockSpec((1, D), lambda i: (0, 0)),
        ],
        out_specs=pl.BlockSpec((RB, D), lambda i: (i, 0)),
        out_shape=jax.ShapeDtypeStruct((N, D), x.dtype),
        compiler_params=pltpu.CompilerParams(
            dimension_semantics=("parallel",),
        ),
    )(x2, tagf, table, gamma2, beta2)
    return out.reshape(B, S, D)


# speedup vs baseline: 1.0616x; 1.0616x over previous
import functools

import jax
import jax.numpy as jnp
from jax.experimental import pallas as pl
from jax.experimental.pallas import tpu as pltpu

EPS = 1e-5


def _qemb_ln_kernel(x_ref, tag_ref, table_ref, gamma_ref, beta_ref, out_ref):
    x = x_ref[...]
    t = tag_ref[...]
    row0 = table_ref[0, :][None, :]
    row1 = table_ref[1, :][None, :]
    emb = jnp.where(t > 0.5, row1, row0)
    y = x + emb
    d = y.shape[-1]
    mean = jnp.sum(y, axis=-1, keepdims=True) * (1.0 / d)
    yc = y - mean
    var = jnp.sum(yc * yc, axis=-1, keepdims=True) * (1.0 / d)
    normed = yc * jax.lax.rsqrt(var + EPS)
    out_ref[...] = normed * gamma_ref[...] + beta_ref[...]


@functools.partial(jax.jit, static_argnames=())
def kernel(x, tags, table, gamma, beta):
    B, S, D = x.shape
    N = B * S
    RB = 1024
    x2 = x.reshape(N, D)
    tagf = tags.reshape(N, 1).astype(jnp.float32)
    gamma2 = gamma.reshape(1, D)
    beta2 = beta.reshape(1, D)

    out = pl.pallas_call(
        _qemb_ln_kernel,
        grid=(N // RB,),
        in_specs=[
            pl.BlockSpec((RB, D), lambda i: (i, 0)),
            pl.BlockSpec((RB, 1), lambda i: (i, 0)),
            pl.BlockSpec((2, D), lambda i: (0, 0)),
            pl.BlockSpec((1, D), lambda i: (0, 0)),
            pl.BlockSpec((1, D), lambda i: (0, 0)),
        ],
        out_specs=pl.BlockSpec((RB, D), lambda i: (i, 0)),
        out_shape=jax.ShapeDtypeStruct((N, D), x.dtype),
        compiler_params=pltpu.CompilerParams(
            dimension_semantics=("parallel",),
        ),
    )(x2, tagf, table, gamma2, beta2)
    return out.reshape(B, S, D)
